# Initial kernel scaffold; baseline (speedup 1.0000x reference)
import jax
import jax.numpy as jnp
from jax.experimental import pallas as pl


def kernel(x, edge_index, W1_l, b1, W1_r, W2_l, b2, W2_r):
    raise NotImplementedError("write your pallas kernel here")


# speedup vs baseline: 4.3304x; 4.3304x over previous
import functools

import jax
import jax.numpy as jnp
from jax import lax
from jax.experimental import pallas as pl
from jax.experimental.pallas import tpu as pltpu
from jax.experimental.pallas import tpu_sc as plsc

N = 10000
D = 128
E = 320000

NC = 2
NS = 16
NW = NC * NS

B = 128
CHUNK = 10112
STEPS = CHUNK // B
EPAD = NW * CHUNK
NPAD = 10112
RPT = NPAD // NS

_mesh = plsc.VectorSubcoreMesh(
    core_axis_name="c", subcore_axis_name="s", num_cores=NC, num_subcores=NS
)


_CH = ((0, 128), (128, 128), (256, 128), (384, 128), (512, 120))


@functools.partial(
    pl.kernel,
    out_type=(
        jax.ShapeDtypeStruct((NC, NPAD, D), jnp.float32),
        jax.ShapeDtypeStruct((NW, NPAD), jnp.float32),
    ),
    mesh=_mesh,
    compiler_params=pltpu.CompilerParams(needs_layout_passes=False),
    scratch_types=(
        pltpu.VMEM((B,), jnp.int32),
        pltpu.VMEM((B,), jnp.int32),
        pltpu.VMEM((B, D), jnp.float32),
        pltpu.VMEM((NPAD,), jnp.float32),
        pltpu.VMEM_SHARED((NPAD, D), jnp.float32),
        pltpu.SemaphoreType.DMA,
    ),
)
def _sc_agg(x_hbm, src_hbm, dst_hbm, z_hbm, z1_hbm, agg_out, cnt_out,
            src_blk, dst_blk, rows, cntv, agg_sh, sem):
    c = lax.axis_index("c")
    s = lax.axis_index("s")
    wid = s * NC + c
    r0 = s * RPT

    pltpu.sync_copy(z_hbm, rows)
    for o, l in _CH:
        pltpu.sync_copy(rows.at[pl.ds(0, l)],
                        agg_sh.at[pl.ds(pl.multiple_of(r0 + o, 8), l)])
    pltpu.sync_copy(z1_hbm, cntv)
    plsc.subcore_barrier()

    ones16 = jnp.ones((16,), jnp.float32)

    def step(i, carry):
        base = pl.multiple_of(wid * CHUNK + i * B, B)
        pltpu.sync_copy(src_hbm.at[pl.ds(base, B)], src_blk)
        pltpu.sync_copy(dst_hbm.at[pl.ds(base, B)], dst_blk)
        pltpu.async_copy(x_hbm.at[src_blk], rows, sem).wait()
        pltpu.sync_copy(rows, agg_sh.at[dst_blk], add=True)
        for k in range(B // 16):
            plsc.addupdate_scatter(cntv, [dst_blk[pl.ds(k * 16, 16)]], ones16)
        return carry

    lax.fori_loop(0, STEPS, step, 0)
    plsc.subcore_barrier()

    for o, l in _CH:
        ro = pl.multiple_of(r0 + o, 8)
        pltpu.sync_copy(agg_sh.at[pl.ds(ro, l)], rows.at[pl.ds(0, l)])
        pltpu.sync_copy(rows.at[pl.ds(0, l)], agg_out.at[c].at[pl.ds(ro, l)])
    pltpu.sync_copy(cntv, cnt_out.at[wid])


BN = 400


def _dense(agg_ref, inv_ref, x, wl_ref, b_ref, wr_ref):
    mean = (agg_ref[0] + agg_ref[1]) * inv_ref[...]
    return (jnp.dot(mean, wl_ref[...], preferred_element_type=jnp.float32)
            + b_ref[...]
            + jnp.dot(x, wr_ref[...], preferred_element_type=jnp.float32))


def _tc1_body(agg_ref, inv_ref, x_ref, wl_ref, b_ref, wr_ref, h_ref):
    z = _dense(agg_ref, inv_ref, x_ref[...], wl_ref, b_ref, wr_ref)
    h_ref[...] = jnp.maximum(z, 0.0)


def _tc2_body(agg_ref, inv_ref, x_ref, wl_ref, b_ref, wr_ref, o_ref):
    z = _dense(agg_ref, inv_ref, x_ref[...], wl_ref, b_ref, wr_ref)
    m = jnp.max(z, axis=1, keepdims=True)
    lse = jnp.log(jnp.sum(jnp.exp(z - m), axis=1, keepdims=True)) + m
    o_ref[...] = z - lse


def _make_tc(body):
    return pl.pallas_call(
        body,
        grid=(N // BN,),
        in_specs=[
            pl.BlockSpec((NC, BN, D), lambda i: (0, i, 0)),
            pl.BlockSpec((BN, 1), lambda i: (i, 0)),
            pl.BlockSpec((BN, D), lambda i: (i, 0)),
            pl.BlockSpec((D, D), lambda i: (0, 0)),
            pl.BlockSpec((1, D), lambda i: (0, 0)),
            pl.BlockSpec((D, D), lambda i: (0, 0)),
        ],
        out_specs=pl.BlockSpec((BN, D), lambda i: (i, 0)),
        out_shape=jax.ShapeDtypeStruct((N, D), jnp.float32),
    )


_tc1 = _make_tc(_tc1_body)
_tc2 = _make_tc(_tc2_body)


def kernel(x, edge_index, W1_l, b1, W1_r, W2_l, b2, W2_r):
    src = edge_index[0].astype(jnp.int32)
    dst = edge_index[1].astype(jnp.int32)
    pad = EPAD - E
    src = jnp.concatenate([src, jnp.zeros((pad,), jnp.int32)])
    dst = jnp.concatenate([dst, jnp.full((pad,), N, jnp.int32)])
    z128 = jnp.zeros((B, D), jnp.float32)
    z1 = jnp.zeros((NPAD,), jnp.float32)

    agg1, cnt = _sc_agg(x, src, dst, z128, z1)
    inv = (1.0 / jnp.maximum(jnp.sum(cnt[:, :N], axis=0), 1.0))[:, None]
    h = _tc1(agg1[:, :N], inv, x, W1_l, b1.reshape(1, D), W1_r)
    agg2, _ = _sc_agg(h, src, dst, z128, z1)
    return _tc2(agg2[:, :N], inv, h, W2_l, b2.reshape(1, D), W2_r)
